# Initial kernel scaffold; baseline (speedup 1.0000x reference)
import jax
import jax.numpy as jnp
from jax.experimental import pallas as pl


def kernel(users, items, user_table, item_table, edge_index, edge_vals):
    raise NotImplementedError("write your pallas kernel here")


# speedup vs baseline: 10.3553x; 10.3553x over previous
import functools

import jax
import jax.numpy as jnp
from jax import lax
from jax.experimental import pallas as pl
from jax.experimental.pallas import tpu as pltpu
from jax.experimental.pallas import tpu_sc as plsc

NU = 50000
NI = 50000
N = NU + NI
E = 1600000
D = 32
H = 16
B = 4096
NC = 2
NS = 16
L = 16

CH = 1024
KROWS = CH // 128
E_PAD = 1638400
ROWS_TOT = E_PAD // 128
ROWS_PER_TILE = ROWS_TOT // NS
CHUNKS = ROWS_PER_TILE // KROWS
N_PAD = 100352
NPT = N_PAD // NS

_mesh = plsc.VectorSubcoreMesh(core_axis_name="c", subcore_axis_name="s")
_params = pltpu.CompilerParams(use_tc_tiling_on_sc=False)

_DN = lax.GatherDimensionNumbers(
    offset_dims=(), collapsed_slice_dims=(0,), start_index_map=(0,))


def _permute(v, idx):
    return lax.gather(v, idx[:, None], _DN, (1,),
                      mode=lax.GatherScatterMode.PROMISE_IN_BOUNDS)


def _lane_splat(v, e):
    return _permute(v, jnp.full((L,), e, jnp.int32))


def _shuf_reduce(v, op):
    lanes = jnp.arange(L, dtype=jnp.int32)
    for sh in (8, 4, 2, 1):
        v = op(v, _permute(v, lanes ^ sh))
    return v


@functools.partial(
    pl.kernel,
    out_type=(
        jax.ShapeDtypeStruct((NC * N_PAD, H), jnp.float32),
        jax.ShapeDtypeStruct((NC * N_PAD, H), jnp.float32),
        jax.ShapeDtypeStruct((NC * N_PAD, H), jnp.float32),
    ),
    mesh=_mesh,
    compiler_params=_params,
    scratch_types=[
        pltpu.VMEM_SHARED((N_PAD, H), jnp.float32),
        pltpu.VMEM((KROWS, 128), jnp.int32),
        pltpu.VMEM((KROWS, 128), jnp.int32),
        pltpu.VMEM((CH,), jnp.float32),
        pltpu.VMEM((CH, H), jnp.float32),
        pltpu.SemaphoreType.DMA,
    ],
)
def _propagate(e0, src2, dst2, vals, zeros, e1, e2, e3,
               acc, src_b, dst_b, val_b, rows_b, sem):
    c = lax.axis_index("c")
    s = lax.axis_index("s")

    def zero_acc():
        pltpu.sync_copy(zeros, acc.at[pl.ds(s * NPT, NPT)])

    def layer(src_ref, out_ref):
        def chunk(t, carry):
            rb = s * ROWS_PER_TILE + t * KROWS
            pltpu.sync_copy(src2.at[c, pl.ds(rb, KROWS)], src_b)
            pltpu.sync_copy(dst2.at[pl.ds(rb, KROWS)], dst_b)
            pltpu.sync_copy(vals.at[pl.ds(rb * 128, CH)], val_b)
            descs = [
                pltpu.async_copy(
                    src_ref.at[src_b.at[k]],
                    rows_b.at[pl.ds(k * 128, 128)], sem)
                for k in range(KROWS)
            ]
            for d in descs:
                d.wait()
            def mul(g, carry2):
                v16 = val_b[pl.ds(g * L, L)]
                for e in range(L):
                    j = g * L + e
                    rows_b[j] = rows_b[j] * _lane_splat(v16, e)
                return carry2
            lax.fori_loop(0, CH // L, mul, 0, unroll=False)
            for k in range(KROWS):
                pltpu.sync_copy(rows_b.at[pl.ds(k * 128, 128)],
                                acc.at[dst_b.at[k]], add=True)
            return carry
        lax.fori_loop(0, CHUNKS, chunk, 0, unroll=False)
        plsc.subcore_barrier()
        pltpu.sync_copy(acc.at[pl.ds(s * NPT, NPT)],
                        out_ref.at[pl.ds(c * N_PAD + s * NPT, NPT)])

    zero_acc()
    plsc.subcore_barrier()
    layer(e0, e1)
    zero_acc()
    plsc.subcore_barrier()
    layer(e1, e2)
    zero_acc()
    plsc.subcore_barrier()
    layer(e2, e3)


BPW = B // (NC * NS)
NBLK = BPW // L


@functools.partial(
    pl.kernel,
    out_type=jax.ShapeDtypeStruct((B,), jnp.float32),
    mesh=_mesh,
    compiler_params=_params,
    scratch_types=[
        pltpu.VMEM((4, BPW), jnp.int32),
        pltpu.VMEM((4, 4, BPW, H), jnp.float32),
        pltpu.VMEM((4, BPW, H), jnp.float32),
        pltpu.VMEM((BPW,), jnp.float32),
        pltpu.SemaphoreType.DMA,
    ],
)
def _head(e0, e1, e2, e3, idx4, gamma, idx_b, g_b, s_b, out_b, sem):
    c = lax.axis_index("c")
    s = lax.axis_index("s")
    wid = c * NS + s
    base = wid * BPW

    pltpu.sync_copy(idx4.at[:, pl.ds(base, BPW)], idx_b)
    descs = []
    for tbl in range(4):
        for li, eref in enumerate((e0, e1, e2, e3)):
            descs.append(pltpu.async_copy(
                eref.at[idx_b.at[tbl]], g_b.at[tbl, li], sem))
    for d in descs:
        d.wait()

    def sum_layers(j, carry):
        for tbl in range(4):
            s_b[tbl, j] = ((g_b[tbl, 0, j] + g_b[tbl, 1, j])
                           + (g_b[tbl, 2, j] + g_b[tbl, 3, j])) * 0.25
        return carry
    lax.fori_loop(0, BPW, sum_layers, 0, unroll=False)

    lanes = lax.iota(jnp.int32, L)

    def blk(b, carry):
        out = jnp.zeros((L,), jnp.float32)
        for e in range(L):
            j = b * L + e
            ulo = s_b[0, j]
            uhi = s_b[1, j]
            ilo = s_b[2, j]
            ihi = s_b[3, j]
            sig_lo = 1.0 / (1.0 + jnp.exp(-ulo))
            sig_hi = 1.0 / (1.0 + jnp.exp(-uhi))
            m = _shuf_reduce(jnp.maximum(ilo, ihi), jnp.maximum)
            exlo = jnp.exp(ilo - m)
            exhi = jnp.exp(ihi - m)
            ssum = _shuf_reduce(exlo + exhi, jnp.add)
            dot = _shuf_reduce(sig_lo * exlo + sig_hi * exhi, jnp.add)
            ge = dot / ssum
            out = jnp.where(lanes == e, ge, out)
        out_b[pl.ds(b * L, L)] = out
        return carry

    lax.fori_loop(0, NBLK, blk, 0, unroll=False)
    pltpu.sync_copy(out_b, gamma.at[pl.ds(base, BPW)])


def kernel(users, items, user_table, item_table, edge_index, edge_vals):
    all_emb = jnp.concatenate([user_table, item_table], axis=0)
    rpad = jnp.zeros((N_PAD - N, H), jnp.float32)
    e0 = jnp.concatenate([all_emb[:, :H], rpad, all_emb[:, H:], rpad], axis=0)

    pad = E_PAD - E
    src = jnp.pad(edge_index[0], (0, pad))
    dst = jnp.pad(edge_index[1], (0, pad))
    vals = jnp.pad(edge_vals, (0, pad))
    src2 = jnp.stack([src, src + N_PAD]).reshape(NC, ROWS_TOT, 128)
    dst2 = dst.reshape(ROWS_TOT, 128)
    zeros = jnp.zeros((NPT, H), jnp.float32)

    e1, e2, e3 = _propagate(e0, src2, dst2, vals, zeros)

    idx4 = jnp.stack([users, users + N_PAD, items + NU, items + NU + N_PAD])
    gamma = _head(e0, e1, e2, e3, idx4)
    return gamma


# speedup vs baseline: 10.8021x; 1.0431x over previous
import functools

import jax
import jax.numpy as jnp
from jax import lax
from jax.experimental import pallas as pl
from jax.experimental.pallas import tpu as pltpu
from jax.experimental.pallas import tpu_sc as plsc

NU = 50000
NI = 50000
N = NU + NI
E = 1600000
D = 32
H = 16
B = 4096
NC = 2
NS = 16
L = 16

CH = 256
KROWS = CH // 128
E_PAD = 1638400
ROWS_TOT = E_PAD // 128
ROWS_PER_TILE = ROWS_TOT // NS
CHUNKS = ROWS_PER_TILE // KROWS
N_PAD = 100352
NPT = N_PAD // NS
NBUF = 4
PIPE_ITERS = CHUNKS // NBUF

_mesh = plsc.VectorSubcoreMesh(core_axis_name="c", subcore_axis_name="s")
_params = pltpu.CompilerParams(use_tc_tiling_on_sc=False)

_DN = lax.GatherDimensionNumbers(
    offset_dims=(), collapsed_slice_dims=(0,), start_index_map=(0,))


def _permute(v, idx):
    return lax.gather(v, idx[:, None], _DN, (1,),
                      mode=lax.GatherScatterMode.PROMISE_IN_BOUNDS)


def _lane_splat(v, e):
    return _permute(v, jnp.full((L,), e, jnp.int32))


def _shuf_reduce(v, op):
    lanes = jnp.arange(L, dtype=jnp.int32)
    for sh in (8, 4, 2, 1):
        v = op(v, _permute(v, lanes ^ sh))
    return v


@functools.partial(
    pl.kernel,
    out_type=(
        jax.ShapeDtypeStruct((NC * N_PAD, H), jnp.float32),
        jax.ShapeDtypeStruct((NC * N_PAD, H), jnp.float32),
        jax.ShapeDtypeStruct((NC * N_PAD, H), jnp.float32),
    ),
    mesh=_mesh,
    compiler_params=_params,
    scratch_types=[
        pltpu.VMEM_SHARED((N_PAD, H), jnp.float32),
        pltpu.VMEM((NBUF, KROWS, 128), jnp.int32),
        pltpu.VMEM((NBUF, KROWS, 128), jnp.int32),
        pltpu.VMEM((NBUF, CH), jnp.float32),
        pltpu.VMEM((NBUF, CH, H), jnp.float32),
        pltpu.SemaphoreType.DMA,
        pltpu.SemaphoreType.DMA,
        pltpu.SemaphoreType.DMA,
        pltpu.SemaphoreType.DMA,
        pltpu.SemaphoreType.DMA,
        pltpu.SemaphoreType.DMA,
        pltpu.SemaphoreType.DMA,
        pltpu.SemaphoreType.DMA,
    ],
)
def _propagate(e0, src2, dst2, vals, zeros, e1, e2, e3,
               acc, src_b, dst_b, val_b, rows_b,
               gs0, gs1, gs2, gs3, ss0, ss1, ss2, ss3):
    c = lax.axis_index("c")
    s = lax.axis_index("s")
    gsem = (gs0, gs1, gs2, gs3)
    ssem = (ss0, ss1, ss2, ss3)

    def zero_acc():
        pltpu.sync_copy(zeros, acc.at[pl.ds(s * NPT, NPT)])

    def layer(src_ref, out_ref):
        def load_and_gather(t, b):
            rb = s * ROWS_PER_TILE + t * KROWS
            pltpu.sync_copy(src2.at[c, pl.ds(rb, KROWS)], src_b.at[b])
            pltpu.sync_copy(dst2.at[pl.ds(rb, KROWS)], dst_b.at[b])
            pltpu.sync_copy(vals.at[pl.ds(rb * 128, CH)], val_b.at[b])
            for k in range(KROWS):
                pltpu.async_copy(src_ref.at[src_b.at[b, k]],
                                 rows_b.at[b, pl.ds(k * 128, 128)], gsem[b])

        def wait_gathers(b):
            for k in range(KROWS):
                pltpu.make_async_copy(src_ref.at[src_b.at[b, k]],
                                      rows_b.at[b, pl.ds(k * 128, 128)],
                                      gsem[b]).wait()

        def mul(b):
            rb_ = rows_b.at[b]
            vb_ = val_b.at[b]
            def mbody(g, carry2):
                v16 = vb_[pl.ds(g * L, L)]
                for e in range(L):
                    j = g * L + e
                    rb_[j] = rb_[j] * _lane_splat(v16, e)
                return carry2
            lax.fori_loop(0, CH // L, mbody, 0, unroll=False)

        def issue_scatters(b):
            for k in range(KROWS):
                pltpu.async_copy(rows_b.at[b, pl.ds(k * 128, 128)],
                                 acc.at[dst_b.at[b, k]], ssem[b], add=True)

        def drain_scatters(b):
            for k in range(KROWS):
                pltpu.make_async_copy(rows_b.at[b, pl.ds(k * 128, 128)],
                                      acc.at[dst_b.at[b, k]], ssem[b]).wait()

        load_and_gather(0, 0)

        def pbody(p, carry):
            for b in range(NBUF):
                t = NBUF * p + b
                b1 = (b + 1) % NBUF
                if b < NBUF - 1:
                    @pl.when(p >= 1)
                    def _():
                        drain_scatters(b1)
                    load_and_gather(t + 1, b1)
                else:
                    drain_scatters(b1)
                    @pl.when(p < PIPE_ITERS - 1)
                    def _():
                        load_and_gather(t + 1, b1)
                wait_gathers(b)
                mul(b)
                issue_scatters(b)
            return carry
        lax.fori_loop(0, PIPE_ITERS, pbody, 0, unroll=False)
        for b in range(1, NBUF):
            drain_scatters(b)
        plsc.subcore_barrier()
        pltpu.sync_copy(acc.at[pl.ds(s * NPT, NPT)],
                        out_ref.at[pl.ds(c * N_PAD + s * NPT, NPT)])

    zero_acc()
    plsc.subcore_barrier()
    layer(e0, e1)
    zero_acc()
    plsc.subcore_barrier()
    layer(e1, e2)
    zero_acc()
    plsc.subcore_barrier()
    layer(e2, e3)


BPW = B // (NC * NS)
NBLK = BPW // L


@functools.partial(
    pl.kernel,
    out_type=jax.ShapeDtypeStruct((B,), jnp.float32),
    mesh=_mesh,
    compiler_params=_params,
    scratch_types=[
        pltpu.VMEM((4, BPW), jnp.int32),
        pltpu.VMEM((4, 4, BPW, H), jnp.float32),
        pltpu.VMEM((4, BPW, H), jnp.float32),
        pltpu.VMEM((BPW,), jnp.float32),
        pltpu.SemaphoreType.DMA,
    ],
)
def _head(e0, e1, e2, e3, idx4, gamma, idx_b, g_b, s_b, out_b, sem):
    c = lax.axis_index("c")
    s = lax.axis_index("s")
    wid = c * NS + s
    base = wid * BPW

    pltpu.sync_copy(idx4.at[:, pl.ds(base, BPW)], idx_b)
    descs = []
    for tbl in range(4):
        for li, eref in enumerate((e0, e1, e2, e3)):
            descs.append(pltpu.async_copy(
                eref.at[idx_b.at[tbl]], g_b.at[tbl, li], sem))
    for d in descs:
        d.wait()

    def sum_layers(j, carry):
        for tbl in range(4):
            s_b[tbl, j] = ((g_b[tbl, 0, j] + g_b[tbl, 1, j])
                           + (g_b[tbl, 2, j] + g_b[tbl, 3, j])) * 0.25
        return carry
    lax.fori_loop(0, BPW, sum_layers, 0, unroll=False)

    lanes = lax.iota(jnp.int32, L)

    def blk(b, carry):
        out = jnp.zeros((L,), jnp.float32)
        for e in range(L):
            j = b * L + e
            ulo = s_b[0, j]
            uhi = s_b[1, j]
            ilo = s_b[2, j]
            ihi = s_b[3, j]
            sig_lo = 1.0 / (1.0 + jnp.exp(-ulo))
            sig_hi = 1.0 / (1.0 + jnp.exp(-uhi))
            m = _shuf_reduce(jnp.maximum(ilo, ihi), jnp.maximum)
            exlo = jnp.exp(ilo - m)
            exhi = jnp.exp(ihi - m)
            ssum = _shuf_reduce(exlo + exhi, jnp.add)
            dot = _shuf_reduce(sig_lo * exlo + sig_hi * exhi, jnp.add)
            ge = dot / ssum
            out = jnp.where(lanes == e, ge, out)
        out_b[pl.ds(b * L, L)] = out
        return carry

    lax.fori_loop(0, NBLK, blk, 0, unroll=False)
    pltpu.sync_copy(out_b, gamma.at[pl.ds(base, BPW)])


def kernel(users, items, user_table, item_table, edge_index, edge_vals):
    all_emb = jnp.concatenate([user_table, item_table], axis=0)
    rpad = jnp.zeros((N_PAD - N, H), jnp.float32)
    e0 = jnp.concatenate([all_emb[:, :H], rpad, all_emb[:, H:], rpad], axis=0)

    pad = E_PAD - E
    src = jnp.pad(edge_index[0], (0, pad))
    dst = jnp.pad(edge_index[1], (0, pad))
    vals = jnp.pad(edge_vals, (0, pad))
    src2 = jnp.stack([src, src + N_PAD]).reshape(NC, ROWS_TOT, 128)
    dst2 = dst.reshape(ROWS_TOT, 128)
    zeros = jnp.zeros((NPT, H), jnp.float32)

    e1, e2, e3 = _propagate(e0, src2, dst2, vals, zeros)

    idx4 = jnp.stack([users, users + N_PAD, items + NU, items + NU + N_PAD])
    gamma = _head(e0, e1, e2, e3, idx4)
    return gamma


# speedup vs baseline: 16.8093x; 1.5561x over previous
import functools

import jax
import jax.numpy as jnp
from jax import lax
from jax.experimental import pallas as pl
from jax.experimental.pallas import tpu as pltpu
from jax.experimental.pallas import tpu_sc as plsc

NU = 50000
NI = 50000
N = NU + NI
E = 1600000
D = 32
H = 16
B = 4096
NC = 2
NS = 16
L = 16

CH = 256
KROWS = CH // 128
E_PAD = 1638400
ROWS_TOT = E_PAD // 128
ROWS_PER_TILE = ROWS_TOT // NS
CHUNKS = ROWS_PER_TILE // KROWS
N_PAD = 100352
NPT = N_PAD // NS
NBUF = 4
PIPE_ITERS = CHUNKS // NBUF

_mesh = plsc.VectorSubcoreMesh(core_axis_name="c", subcore_axis_name="s")
_params = pltpu.CompilerParams(use_tc_tiling_on_sc=False)

_DN = lax.GatherDimensionNumbers(
    offset_dims=(), collapsed_slice_dims=(0,), start_index_map=(0,))


def _permute(v, idx):
    return lax.gather(v, idx[:, None], _DN, (1,),
                      mode=lax.GatherScatterMode.PROMISE_IN_BOUNDS)


def _lane_splat(v, e):
    return _permute(v, jnp.full((L,), e, jnp.int32))


def _shuf_reduce(v, op):
    lanes = jnp.arange(L, dtype=jnp.int32)
    for sh in (8, 4, 2, 1):
        v = op(v, _permute(v, lanes ^ sh))
    return v


@functools.partial(
    pl.kernel,
    out_type=(
        jax.ShapeDtypeStruct((NC * N_PAD, H), jnp.float32),
        jax.ShapeDtypeStruct((NC * N_PAD, H), jnp.float32),
        jax.ShapeDtypeStruct((NC * N_PAD, H), jnp.float32),
    ),
    mesh=_mesh,
    compiler_params=_params,
    scratch_types=[
        pltpu.VMEM_SHARED((N_PAD, H), jnp.float32),
        pltpu.VMEM((NBUF, KROWS, 128), jnp.int32),
        pltpu.VMEM((NBUF, KROWS, 128), jnp.int32),
        pltpu.VMEM((NBUF, CH), jnp.float32),
        pltpu.VMEM((NBUF, CH, H), jnp.float32),
        pltpu.SemaphoreType.DMA,
        pltpu.SemaphoreType.DMA,
        pltpu.SemaphoreType.DMA,
        pltpu.SemaphoreType.DMA,
        pltpu.SemaphoreType.DMA,
        pltpu.SemaphoreType.DMA,
        pltpu.SemaphoreType.DMA,
        pltpu.SemaphoreType.DMA,
        pltpu.SemaphoreType.DMA,
        pltpu.SemaphoreType.DMA,
        pltpu.SemaphoreType.DMA,
        pltpu.SemaphoreType.DMA,
    ],
)
def _propagate(e0, src2, dst2, vals, zeros, e1, e2, e3,
               acc, src_b, dst_b, val_b, rows_b,
               gs0, gs1, gs2, gs3, ss0, ss1, ss2, ss3,
               es0, es1, es2, es3):
    c = lax.axis_index("c")
    s = lax.axis_index("s")
    gsem = (gs0, gs1, gs2, gs3)
    ssem = (ss0, ss1, ss2, ss3)
    esem = (es0, es1, es2, es3)

    def zero_acc():
        pltpu.sync_copy(zeros, acc.at[pl.ds(s * NPT, NPT)])

    def layer(src_ref, out_ref):
        def edge_copies(t, b):
            rb = s * ROWS_PER_TILE + t * KROWS
            return (
                (src2.at[c, pl.ds(rb, KROWS)], src_b.at[b]),
                (dst2.at[pl.ds(rb, KROWS)], dst_b.at[b]),
                (vals.at[pl.ds(rb * 128, CH)], val_b.at[b]),
            )

        def issue_edge_loads(t, b):
            for sr, dr in edge_copies(t, b):
                pltpu.async_copy(sr, dr, esem[b])

        def wait_edge_loads(t, b):
            for sr, dr in edge_copies(t, b):
                pltpu.make_async_copy(sr, dr, esem[b]).wait()

        def issue_gathers(b):
            for k in range(KROWS):
                pltpu.async_copy(src_ref.at[src_b.at[b, k]],
                                 rows_b.at[b, pl.ds(k * 128, 128)], gsem[b])

        def wait_gathers(b):
            for k in range(KROWS):
                pltpu.make_async_copy(src_ref.at[src_b.at[b, k]],
                                      rows_b.at[b, pl.ds(k * 128, 128)],
                                      gsem[b]).wait()

        def mul(b):
            rb_ = rows_b.at[b]
            vb_ = val_b.at[b]
            def mbody(g, carry2):
                v16 = vb_[pl.ds(g * L, L)]
                for e in range(L):
                    j = g * L + e
                    rb_[j] = rb_[j] * _lane_splat(v16, e)
                return carry2
            lax.fori_loop(0, CH // L, mbody, 0, unroll=False)

        def issue_scatters(b):
            for k in range(KROWS):
                pltpu.async_copy(rows_b.at[b, pl.ds(k * 128, 128)],
                                 acc.at[dst_b.at[b, k]], ssem[b], add=True)

        def drain_scatters(b):
            for k in range(KROWS):
                pltpu.make_async_copy(rows_b.at[b, pl.ds(k * 128, 128)],
                                      acc.at[dst_b.at[b, k]], ssem[b]).wait()

        issue_edge_loads(0, 0)
        wait_edge_loads(0, 0)
        issue_gathers(0)
        issue_edge_loads(1, 1)

        def pbody(p, carry):
            for b in range(NBUF):
                t = NBUF * p + b
                b1 = (b + 1) % NBUF
                b2 = (b + 2) % NBUF
                if b >= 2:
                    drain_scatters(b2)
                else:
                    @pl.when(p >= 1)
                    def _():
                        drain_scatters(b2)
                if b < NBUF - 1:
                    wait_edge_loads(t + 1, b1)
                    issue_gathers(b1)
                    if b < NBUF - 2:
                        issue_edge_loads(t + 2, b2)
                    else:
                        @pl.when(p < PIPE_ITERS - 1)
                        def _():
                            issue_edge_loads(t + 2, b2)
                else:
                    @pl.when(p < PIPE_ITERS - 1)
                    def _():
                        wait_edge_loads(t + 1, b1)
                        issue_gathers(b1)
                        issue_edge_loads(t + 2, b2)
                wait_gathers(b)
                mul(b)
                issue_scatters(b)
            return carry
        lax.fori_loop(0, PIPE_ITERS, pbody, 0, unroll=False)
        drain_scatters(2)
        drain_scatters(3)
        plsc.subcore_barrier()
        pltpu.sync_copy(acc.at[pl.ds(s * NPT, NPT)],
                        out_ref.at[pl.ds(c * N_PAD + s * NPT, NPT)])

    zero_acc()
    plsc.subcore_barrier()
    layer(e0, e1)
    zero_acc()
    plsc.subcore_barrier()
    layer(e1, e2)
    zero_acc()
    plsc.subcore_barrier()
    layer(e2, e3)


BPW = B // (NC * NS)
NBLK = BPW // L


@functools.partial(
    pl.kernel,
    out_type=jax.ShapeDtypeStruct((B,), jnp.float32),
    mesh=_mesh,
    compiler_params=_params,
    scratch_types=[
        pltpu.VMEM((4, BPW), jnp.int32),
        pltpu.VMEM((4, 4, BPW, H), jnp.float32),
        pltpu.VMEM((4, BPW, H), jnp.float32),
        pltpu.VMEM((BPW,), jnp.float32),
        pltpu.SemaphoreType.DMA,
    ],
)
def _head(e0, e1, e2, e3, idx4, gamma, idx_b, g_b, s_b, out_b, sem):
    c = lax.axis_index("c")
    s = lax.axis_index("s")
    wid = c * NS + s
    base = wid * BPW

    pltpu.sync_copy(idx4.at[:, pl.ds(base, BPW)], idx_b)
    descs = []
    for tbl in range(4):
        for li, eref in enumerate((e0, e1, e2, e3)):
            descs.append(pltpu.async_copy(
                eref.at[idx_b.at[tbl]], g_b.at[tbl, li], sem))
    for d in descs:
        d.wait()

    def sum_layers(j, carry):
        for tbl in range(4):
            s_b[tbl, j] = ((g_b[tbl, 0, j] + g_b[tbl, 1, j])
                           + (g_b[tbl, 2, j] + g_b[tbl, 3, j])) * 0.25
        return carry
    lax.fori_loop(0, BPW, sum_layers, 0, unroll=False)

    lanes = lax.iota(jnp.int32, L)

    def blk(b, carry):
        out = jnp.zeros((L,), jnp.float32)
        for e in range(L):
            j = b * L + e
            ulo = s_b[0, j]
            uhi = s_b[1, j]
            ilo = s_b[2, j]
            ihi = s_b[3, j]
            sig_lo = 1.0 / (1.0 + jnp.exp(-ulo))
            sig_hi = 1.0 / (1.0 + jnp.exp(-uhi))
            m = _shuf_reduce(jnp.maximum(ilo, ihi), jnp.maximum)
            exlo = jnp.exp(ilo - m)
            exhi = jnp.exp(ihi - m)
            ssum = _shuf_reduce(exlo + exhi, jnp.add)
            dot = _shuf_reduce(sig_lo * exlo + sig_hi * exhi, jnp.add)
            ge = dot / ssum
            out = jnp.where(lanes == e, ge, out)
        out_b[pl.ds(b * L, L)] = out
        return carry

    lax.fori_loop(0, NBLK, blk, 0, unroll=False)
    pltpu.sync_copy(out_b, gamma.at[pl.ds(base, BPW)])


def kernel(users, items, user_table, item_table, edge_index, edge_vals):
    all_emb = jnp.concatenate([user_table, item_table], axis=0)
    rpad = jnp.zeros((N_PAD - N, H), jnp.float32)
    e0 = jnp.concatenate([all_emb[:, :H], rpad, all_emb[:, H:], rpad], axis=0)

    pad = E_PAD - E
    src = jnp.pad(edge_index[0], (0, pad))
    dst = jnp.pad(edge_index[1], (0, pad))
    vals = jnp.pad(edge_vals, (0, pad))
    src2 = jnp.stack([src, src + N_PAD]).reshape(NC, ROWS_TOT, 128)
    dst2 = dst.reshape(ROWS_TOT, 128)
    zeros = jnp.zeros((NPT, H), jnp.float32)

    e1, e2, e3 = _propagate(e0, src2, dst2, vals, zeros)

    idx4 = jnp.stack([users, users + N_PAD, items + NU, items + NU + N_PAD])
    gamma = _head(e0, e1, e2, e3, idx4)
    return gamma


# speedup vs baseline: 17.0045x; 1.0116x over previous
import functools

import jax
import jax.numpy as jnp
from jax import lax
from jax.experimental import pallas as pl
from jax.experimental.pallas import tpu as pltpu
from jax.experimental.pallas import tpu_sc as plsc

NU = 50000
NI = 50000
N = NU + NI
E = 1600000
D = 32
H = 16
B = 4096
NC = 2
NS = 16
L = 16

CH = 256
KROWS = CH // 128
E_PAD = 1638400
ROWS_TOT = E_PAD // 128
ROWS_PER_TILE = ROWS_TOT // NS
CHUNKS = ROWS_PER_TILE // KROWS
N_PAD = 100352
NPT = N_PAD // NS
NBUF = 4
EBUF = 8
PIPE_ITERS = CHUNKS // EBUF

_mesh = plsc.VectorSubcoreMesh(core_axis_name="c", subcore_axis_name="s")
_params = pltpu.CompilerParams(use_tc_tiling_on_sc=False)

_DN = lax.GatherDimensionNumbers(
    offset_dims=(), collapsed_slice_dims=(0,), start_index_map=(0,))


def _permute(v, idx):
    return lax.gather(v, idx[:, None], _DN, (1,),
                      mode=lax.GatherScatterMode.PROMISE_IN_BOUNDS)


def _lane_splat(v, e):
    return _permute(v, jnp.full((L,), e, jnp.int32))


def _shuf_reduce(v, op):
    lanes = jnp.arange(L, dtype=jnp.int32)
    for sh in (8, 4, 2, 1):
        v = op(v, _permute(v, lanes ^ sh))
    return v


@functools.partial(
    pl.kernel,
    out_type=(
        jax.ShapeDtypeStruct((NC * N_PAD, H), jnp.float32),
        jax.ShapeDtypeStruct((NC * N_PAD, H), jnp.float32),
        jax.ShapeDtypeStruct((NC * N_PAD, H), jnp.float32),
    ),
    mesh=_mesh,
    compiler_params=_params,
    scratch_types=[
        pltpu.VMEM_SHARED((N_PAD, H), jnp.float32),
        pltpu.VMEM((EBUF, KROWS, 128), jnp.int32),
        pltpu.VMEM((EBUF, KROWS, 128), jnp.int32),
        pltpu.VMEM((EBUF, CH), jnp.float32),
        pltpu.VMEM((NBUF, CH, H), jnp.float32),
        pltpu.SemaphoreType.DMA,
        pltpu.SemaphoreType.DMA,
        pltpu.SemaphoreType.DMA,
        pltpu.SemaphoreType.DMA,
        pltpu.SemaphoreType.DMA,
        pltpu.SemaphoreType.DMA,
        pltpu.SemaphoreType.DMA,
        pltpu.SemaphoreType.DMA,
        pltpu.SemaphoreType.DMA,
        pltpu.SemaphoreType.DMA,
        pltpu.SemaphoreType.DMA,
        pltpu.SemaphoreType.DMA,
        pltpu.SemaphoreType.DMA,
        pltpu.SemaphoreType.DMA,
        pltpu.SemaphoreType.DMA,
        pltpu.SemaphoreType.DMA,
    ],
)
def _propagate(e0, src2, dst2, vals, zeros, e1, e2, e3,
               acc, src_b, dst_b, val_b, rows_b,
               gs0, gs1, gs2, gs3, ss0, ss1, ss2, ss3,
               es0, es1, es2, es3, es4, es5, es6, es7):
    c = lax.axis_index("c")
    s = lax.axis_index("s")
    gsem = (gs0, gs1, gs2, gs3)
    ssem = (ss0, ss1, ss2, ss3)
    esem = (es0, es1, es2, es3, es4, es5, es6, es7)

    def zero_acc():
        pltpu.sync_copy(zeros, acc.at[pl.ds(s * NPT, NPT)])

    def layer(src_ref, out_ref):
        def edge_copies(t, e):
            rb = s * ROWS_PER_TILE + t * KROWS
            return (
                (src2.at[c, pl.ds(rb, KROWS)], src_b.at[e]),
                (dst2.at[pl.ds(rb, KROWS)], dst_b.at[e]),
                (vals.at[pl.ds(rb * 128, CH)], val_b.at[e]),
            )

        def issue_edge_loads(t, e):
            for sr, dr in edge_copies(t, e):
                pltpu.async_copy(sr, dr, esem[e])

        def wait_edge_loads(t, e):
            for sr, dr in edge_copies(t, e):
                pltpu.make_async_copy(sr, dr, esem[e]).wait()

        def issue_gathers(r, e):
            for k in range(KROWS):
                pltpu.async_copy(src_ref.at[src_b.at[e, k]],
                                 rows_b.at[r, pl.ds(k * 128, 128)], gsem[r])

        def wait_gathers(r, e):
            for k in range(KROWS):
                pltpu.make_async_copy(src_ref.at[src_b.at[e, k]],
                                      rows_b.at[r, pl.ds(k * 128, 128)],
                                      gsem[r]).wait()

        def mul(r, e):
            rb_ = rows_b.at[r]
            vb_ = val_b.at[e]
            def mbody(g, carry2):
                v16 = vb_[pl.ds(g * L, L)]
                for j0 in range(L):
                    j = g * L + j0
                    rb_[j] = rb_[j] * _lane_splat(v16, j0)
                return carry2
            lax.fori_loop(0, CH // L, mbody, 0, unroll=False)

        def issue_scatters(r, e):
            for k in range(KROWS):
                pltpu.async_copy(rows_b.at[r, pl.ds(k * 128, 128)],
                                 acc.at[dst_b.at[e, k]], ssem[r], add=True)

        def drain_scatters(r, e):
            for k in range(KROWS):
                pltpu.make_async_copy(rows_b.at[r, pl.ds(k * 128, 128)],
                                      acc.at[dst_b.at[e, k]], ssem[r]).wait()

        issue_edge_loads(0, 0)
        issue_edge_loads(1, 1)
        issue_edge_loads(2, 2)
        wait_edge_loads(0, 0)
        issue_gathers(0, 0)
        wait_edge_loads(1, 1)
        issue_gathers(1, 1)

        def pbody(p, carry):
            for q in range(EBUF):
                t = EBUF * p + q
                r = q % NBUF
                r2 = (q + 2) % NBUF
                e = q
                e2 = (q + 2) % EBUF
                e3 = (q + 3) % EBUF
                em2 = (q - 2) % EBUF
                if q >= 2:
                    drain_scatters(r2, em2)
                else:
                    @pl.when(p >= 1)
                    def _():
                        drain_scatters(r2, em2)
                if q < EBUF - 2:
                    wait_edge_loads(t + 2, e2)
                    issue_gathers(r2, e2)
                else:
                    @pl.when(p < PIPE_ITERS - 1)
                    def _():
                        wait_edge_loads(t + 2, e2)
                        issue_gathers(r2, e2)
                if q < EBUF - 3:
                    issue_edge_loads(t + 3, e3)
                else:
                    @pl.when(p < PIPE_ITERS - 1)
                    def _():
                        issue_edge_loads(t + 3, e3)
                wait_gathers(r, e)
                mul(r, e)
                issue_scatters(r, e)
            return carry
        lax.fori_loop(0, PIPE_ITERS, pbody, 0, unroll=False)
        drain_scatters((CHUNKS - 2) % NBUF, (CHUNKS - 2) % EBUF)
        drain_scatters((CHUNKS - 1) % NBUF, (CHUNKS - 1) % EBUF)
        plsc.subcore_barrier()
        pltpu.sync_copy(acc.at[pl.ds(s * NPT, NPT)],
                        out_ref.at[pl.ds(c * N_PAD + s * NPT, NPT)])

    zero_acc()
    plsc.subcore_barrier()
    layer(e0, e1)
    zero_acc()
    plsc.subcore_barrier()
    layer(e1, e2)
    zero_acc()
    plsc.subcore_barrier()
    layer(e2, e3)


BPW = B // (NC * NS)
NBLK = BPW // L


@functools.partial(
    pl.kernel,
    out_type=jax.ShapeDtypeStruct((B,), jnp.float32),
    mesh=_mesh,
    compiler_params=_params,
    scratch_types=[
        pltpu.VMEM((4, BPW), jnp.int32),
        pltpu.VMEM((4, 4, BPW, H), jnp.float32),
        pltpu.VMEM((4, BPW, H), jnp.float32),
        pltpu.VMEM((BPW,), jnp.float32),
        pltpu.SemaphoreType.DMA,
    ],
)
def _head(e0, e1, e2, e3, idx4, gamma, idx_b, g_b, s_b, out_b, sem):
    c = lax.axis_index("c")
    s = lax.axis_index("s")
    wid = c * NS + s
    base = wid * BPW

    pltpu.sync_copy(idx4.at[:, pl.ds(base, BPW)], idx_b)
    descs = []
    for tbl in range(4):
        for li, eref in enumerate((e0, e1, e2, e3)):
            descs.append(pltpu.async_copy(
                eref.at[idx_b.at[tbl]], g_b.at[tbl, li], sem))
    for d in descs:
        d.wait()

    def sum_layers(j, carry):
        for tbl in range(4):
            s_b[tbl, j] = ((g_b[tbl, 0, j] + g_b[tbl, 1, j])
                           + (g_b[tbl, 2, j] + g_b[tbl, 3, j])) * 0.25
        return carry
    lax.fori_loop(0, BPW, sum_layers, 0, unroll=False)

    lanes = lax.iota(jnp.int32, L)

    def blk(b, carry):
        out = jnp.zeros((L,), jnp.float32)
        for e in range(L):
            j = b * L + e
            ulo = s_b[0, j]
            uhi = s_b[1, j]
            ilo = s_b[2, j]
            ihi = s_b[3, j]
            sig_lo = 1.0 / (1.0 + jnp.exp(-ulo))
            sig_hi = 1.0 / (1.0 + jnp.exp(-uhi))
            m = _shuf_reduce(jnp.maximum(ilo, ihi), jnp.maximum)
            exlo = jnp.exp(ilo - m)
            exhi = jnp.exp(ihi - m)
            ssum = _shuf_reduce(exlo + exhi, jnp.add)
            dot = _shuf_reduce(sig_lo * exlo + sig_hi * exhi, jnp.add)
            ge = dot / ssum
            out = jnp.where(lanes == e, ge, out)
        out_b[pl.ds(b * L, L)] = out
        return carry

    lax.fori_loop(0, NBLK, blk, 0, unroll=False)
    pltpu.sync_copy(out_b, gamma.at[pl.ds(base, BPW)])


def kernel(users, items, user_table, item_table, edge_index, edge_vals):
    all_emb = jnp.concatenate([user_table, item_table], axis=0)
    rpad = jnp.zeros((N_PAD - N, H), jnp.float32)
    e0 = jnp.concatenate([all_emb[:, :H], rpad, all_emb[:, H:], rpad], axis=0)

    pad = E_PAD - E
    src = jnp.pad(edge_index[0], (0, pad))
    dst = jnp.pad(edge_index[1], (0, pad))
    vals = jnp.pad(edge_vals, (0, pad))
    src2 = jnp.stack([src, src + N_PAD]).reshape(NC, ROWS_TOT, 128)
    dst2 = dst.reshape(ROWS_TOT, 128)
    zeros = jnp.zeros((NPT, H), jnp.float32)

    e1, e2, e3 = _propagate(e0, src2, dst2, vals, zeros)

    idx4 = jnp.stack([users, users + N_PAD, items + NU, items + NU + N_PAD])
    gamma = _head(e0, e1, e2, e3, idx4)
    return gamma
